# Initial kernel scaffold; baseline (speedup 1.0000x reference)
import jax
import jax.numpy as jnp
from jax.experimental import pallas as pl


def kernel(xyz, sparse_xyz, sparse_frame):
    raise NotImplementedError("write your pallas kernel here")


# speedup vs baseline: 29.0443x; 29.0443x over previous
import jax
import jax.numpy as jnp
from jax.experimental import pallas as pl


def _body(xq_ref, sx_ref, sf_ref, out_ref):
    S, _ = sx_ref.shape
    TN = xq_ref.shape[1]
    f32 = jnp.float32

    sxb = sx_ref[:, :].astype(jnp.bfloat16)
    xqb = xq_ref[:, :].astype(jnp.bfloat16)
    dot = jax.lax.dot_general(
        sxb, xqb, (((1,), (0,)), ((), ())), preferred_element_type=f32
    )
    xnorm = jnp.zeros((1, TN), f32)
    snorm = jnp.zeros((S, 1), f32)
    for c in range(3):
        xc = xq_ref[c : c + 1, :]
        sc = sx_ref[:, c : c + 1]
        xnorm = xnorm + xc * xc
        snorm = snorm + sc * sc
    d_sel = (-2.0 * dot + xnorm) + snorm

    d_exact = jnp.zeros((S, TN), f32)
    for c in range(3):
        diff = sx_ref[:, c : c + 1] - xq_ref[c : c + 1, :]
        d_exact = d_exact + diff * diff

    rows = jax.lax.broadcasted_iota(jnp.int32, (S, TN), 0)
    exact_vals = []
    for k in range(3):
        am = jnp.argmin(d_sel, axis=0)[None, :]
        hit = rows == am
        exact_vals.append(jnp.sum(jnp.where(hit, d_exact, 0.0), axis=0, keepdims=True))
        if k < 2:
            d_sel = jnp.where(hit, jnp.float32(jnp.inf), d_sel)

    invs = [1.0 / jnp.maximum(jnp.sqrt(v), 1e-10) for v in exact_vals]
    norm = invs[0] + invs[1] + invs[2]

    out = jnp.zeros((S, TN), f32)
    for k in range(3):
        w = invs[k] / norm
        out = out + sf_ref[:, k : k + 1] * w
    out_ref[:, :] = out


@jax.jit
def kernel(xyz, sparse_xyz, sparse_frame):
    B, C, N = xyz.shape
    S = sparse_xyz.shape[2]
    TN = 256

    xq = xyz[0]
    sx = jnp.transpose(sparse_xyz[0])
    sf = jnp.transpose(sparse_frame[0])

    out = pl.pallas_call(
        _body,
        grid=(N // TN,),
        in_specs=[
            pl.BlockSpec((3, TN), lambda i: (0, i)),
            pl.BlockSpec((S, 3), lambda i: (0, 0)),
            pl.BlockSpec((S, 3), lambda i: (0, 0)),
        ],
        out_specs=pl.BlockSpec((S, TN), lambda i: (0, i)),
        out_shape=jax.ShapeDtypeStruct((S, N), jnp.float32),
    )(xq, sx, sf)
    return out[None]


# speedup vs baseline: 42.9926x; 1.4802x over previous
import jax
import jax.numpy as jnp
from jax.experimental import pallas as pl
from jax.experimental.pallas import tpu as pltpu


def _body(xq_ref, xsel_ref, xpair_ref, sxsel_ref, sxpair_ref, sfpair_ref,
          sx_ref, out_ref, snorm_ref):
    S = sx_ref.shape[0]
    TN = xq_ref.shape[1]
    f32 = jnp.float32

    @pl.when(pl.program_id(0) == 0)
    def _():
        sn = jnp.zeros((S, 1), f32)
        for c in range(3):
            sc = sx_ref[:, c : c + 1]
            sn = sn + sc * sc
        snorm_ref[:, :] = sn

    snorm = snorm_ref[:, :]
    xnorm = jnp.zeros((1, TN), f32)
    for c in range(3):
        xc = xq_ref[c : c + 1, :]
        xnorm = xnorm + xc * xc

    dotb = jax.lax.dot_general(
        sxsel_ref[:, :], xsel_ref[:, :], (((1,), (0,)), ((), ())),
        preferred_element_type=f32,
    )
    d_sel = (-2.0 * dotb + xnorm) + snorm

    dote = jax.lax.dot_general(
        sxpair_ref[:, :], xpair_ref[:, :], (((1,), (0,)), ((), ())),
        preferred_element_type=f32,
    )
    d_exact = (xnorm + snorm) - (dote + dote)

    big = jnp.float32(jnp.inf)
    exact_vals = []
    for k in range(3):
        v = jnp.min(d_sel, axis=0, keepdims=True)
        eq = d_sel == v
        exact_vals.append(
            jnp.min(jnp.where(eq, d_exact, big), axis=0, keepdims=True)
        )
        if k < 2:
            d_sel = jnp.where(eq, big, d_sel)

    invs = [
        1.0 / jnp.maximum(jnp.sqrt(jnp.maximum(v, 0.0)), 1e-10)
        for v in exact_vals
    ]
    norm = invs[0] + invs[1] + invs[2]

    bf16 = jnp.bfloat16
    w = [iv / norm for iv in invs]
    wh = [x.astype(bf16) for x in w]
    wl = [(x - y.astype(f32)).astype(bf16) for x, y in zip(w, wh)]
    wmat = jnp.concatenate(wh + wh + wl, axis=0)
    out_ref[:, :] = jax.lax.dot_general(
        sfpair_ref[:, :], wmat, (((1,), (0,)), ((), ())),
        preferred_element_type=f32,
    )


def _split_bf16(a):
    hi_f32 = jax.lax.reduce_precision(a, exponent_bits=8, mantissa_bits=7)
    hi = hi_f32.astype(jnp.bfloat16)
    lo = (a - hi_f32).astype(jnp.bfloat16)
    return hi, lo


@jax.jit
def kernel(xyz, sparse_xyz, sparse_frame):
    B, C, N = xyz.shape
    S = sparse_xyz.shape[2]
    TN = 256
    f32 = jnp.float32

    xq = xyz[0]
    sx = jnp.transpose(sparse_xyz[0])
    sf = jnp.transpose(sparse_frame[0])

    xh, xl = _split_bf16(xq)
    sxh, sxl = _split_bf16(sx)
    sfh, sfl = _split_bf16(sf)
    xsel = xh
    sxsel = sxh
    xpair = jnp.concatenate([xh, xh, xl], axis=0)
    sxpair = jnp.concatenate([sxh, sxl, sxh], axis=1)
    sfpair = jnp.concatenate([sfh, sfl, sfh], axis=1)

    out = pl.pallas_call(
        _body,
        grid=(N // TN,),
        in_specs=[
            pl.BlockSpec((3, TN), lambda i: (0, i)),
            pl.BlockSpec((3, TN), lambda i: (0, i)),
            pl.BlockSpec((9, TN), lambda i: (0, i)),
            pl.BlockSpec((S, 3), lambda i: (0, 0)),
            pl.BlockSpec((S, 9), lambda i: (0, 0)),
            pl.BlockSpec((S, 9), lambda i: (0, 0)),
            pl.BlockSpec((S, 3), lambda i: (0, 0)),
        ],
        out_specs=pl.BlockSpec((S, TN), lambda i: (0, i)),
        out_shape=jax.ShapeDtypeStruct((S, N), f32),
        scratch_shapes=[pltpu.VMEM((S, 1), f32)],
    )(xq, xsel, xpair, sxsel, sxpair, sfpair, sx)
    return out[None]


# speedup vs baseline: 44.5045x; 1.0352x over previous
import jax
import jax.numpy as jnp
from jax.experimental import pallas as pl
from jax.experimental.pallas import tpu as pltpu


def _body(xq_ref, xsel_ref, xpair_ref, sxsel_ref, sxpair_ref, sfpair_ref,
          sx_ref, out_ref, snorm_ref):
    S = sx_ref.shape[0]
    TN = xq_ref.shape[1]
    f32 = jnp.float32

    @pl.when(pl.program_id(0) == 0)
    def _():
        sn = jnp.zeros((S, 1), f32)
        for c in range(3):
            sc = sx_ref[:, c : c + 1]
            sn = sn + sc * sc
        snorm_ref[:, :] = sn

    snorm = snorm_ref[:, :]
    xnorm = jnp.zeros((1, TN), f32)
    for c in range(3):
        xc = xq_ref[c : c + 1, :]
        xnorm = xnorm + xc * xc

    dotb = jax.lax.dot_general(
        sxsel_ref[:, :], xsel_ref[:, :], (((1,), (0,)), ((), ())),
        preferred_element_type=f32,
    )
    d_sel = (-2.0 * dotb + xnorm) + snorm

    dcorr = jax.lax.dot_general(
        sxpair_ref[:, :], xpair_ref[:, :], (((1,), (0,)), ((), ())),
        preferred_element_type=f32,
    )
    d_exact = d_sel - (dcorr + dcorr)

    big = jnp.float32(jnp.inf)
    exact_vals = []
    for k in range(3):
        v = jnp.min(d_sel, axis=0, keepdims=True)
        eq = d_sel == v
        exact_vals.append(
            jnp.min(jnp.where(eq, d_exact, big), axis=0, keepdims=True)
        )
        if k < 2:
            d_sel = jnp.where(eq, big, d_sel)

    invs = [
        1.0 / jnp.maximum(jnp.sqrt(jnp.maximum(v, 0.0)), 1e-10)
        for v in exact_vals
    ]
    norm = invs[0] + invs[1] + invs[2]

    bf16 = jnp.bfloat16
    w = [iv / norm for iv in invs]
    wh = [x.astype(bf16) for x in w]
    wl = [(x - y.astype(f32)).astype(bf16) for x, y in zip(w, wh)]
    wmat = jnp.concatenate(wh + wh + wl, axis=0)
    out_ref[:, :] = jax.lax.dot_general(
        sfpair_ref[:, :], wmat, (((1,), (0,)), ((), ())),
        preferred_element_type=f32,
    )


def _split_bf16(a):
    hi_f32 = jax.lax.reduce_precision(a, exponent_bits=8, mantissa_bits=7)
    hi = hi_f32.astype(jnp.bfloat16)
    lo = (a - hi_f32).astype(jnp.bfloat16)
    return hi, lo


@jax.jit
def kernel(xyz, sparse_xyz, sparse_frame):
    B, C, N = xyz.shape
    S = sparse_xyz.shape[2]
    TN = 256
    f32 = jnp.float32

    xq = xyz[0]
    sx = jnp.transpose(sparse_xyz[0])
    sf = jnp.transpose(sparse_frame[0])

    xh, xl = _split_bf16(xq)
    sxh, sxl = _split_bf16(sx)
    sfh, sfl = _split_bf16(sf)
    xsel = xh
    sxsel = sxh
    xpair = jnp.concatenate([xh, xl], axis=0)
    sxpair = jnp.concatenate([sxl, sxh], axis=1)
    sfpair = jnp.concatenate([sfh, sfl, sfh], axis=1)

    out = pl.pallas_call(
        _body,
        grid=(N // TN,),
        in_specs=[
            pl.BlockSpec((3, TN), lambda i: (0, i)),
            pl.BlockSpec((3, TN), lambda i: (0, i)),
            pl.BlockSpec((6, TN), lambda i: (0, i)),
            pl.BlockSpec((S, 3), lambda i: (0, 0)),
            pl.BlockSpec((S, 6), lambda i: (0, 0)),
            pl.BlockSpec((S, 9), lambda i: (0, 0)),
            pl.BlockSpec((S, 3), lambda i: (0, 0)),
        ],
        out_specs=pl.BlockSpec((S, TN), lambda i: (0, i)),
        out_shape=jax.ShapeDtypeStruct((S, N), f32),
        scratch_shapes=[pltpu.VMEM((S, 1), f32)],
    )(xq, xsel, xpair, sxsel, sxpair, sfpair, sx)
    return out[None]


# speedup vs baseline: 54.6195x; 1.2273x over previous
import jax
import jax.numpy as jnp
from jax.experimental import pallas as pl
from jax.experimental.pallas import tpu as pltpu


def _body(xq_ref, xsel_ref, xpair_ref, sxsel_ref, sxpair_ref, sfpair_ref,
          sx_ref, out_ref, snorm_ref):
    S = sx_ref.shape[0]
    TN = xq_ref.shape[1]
    f32 = jnp.float32

    @pl.when(pl.program_id(0) == 0)
    def _():
        sn = jnp.zeros((S, 1), f32)
        for c in range(3):
            sc = sx_ref[:, c : c + 1]
            sn = sn + sc * sc
        snorm_ref[:, :] = sn

    snorm = snorm_ref[:, :]
    xnorm = jnp.zeros((1, TN), f32)
    for c in range(3):
        xc = xq_ref[c : c + 1, :]
        xnorm = xnorm + xc * xc

    dotb = jax.lax.dot_general(
        sxsel_ref[:, :], xsel_ref[:, :], (((1,), (0,)), ((), ())),
        preferred_element_type=f32,
    )
    d_sel = (-2.0 * dotb + xnorm) + snorm

    dcorr = jax.lax.dot_general(
        sxpair_ref[:, :], xpair_ref[:, :], (((1,), (0,)), ((), ())),
        preferred_element_type=f32,
    )
    d_exact = d_sel - (dcorr + dcorr)

    big = jnp.float32(jnp.inf)
    exact_vals = []
    for k in range(3):
        v = jnp.min(d_sel, axis=0, keepdims=True)
        eq = d_sel == v
        exact_vals.append(
            jnp.min(jnp.where(eq, d_exact, big), axis=0, keepdims=True)
        )
        if k < 2:
            d_sel = jnp.where(eq, big, d_sel)

    invs = [
        1.0 / jnp.maximum(jnp.sqrt(jnp.maximum(v, 0.0)), 1e-10)
        for v in exact_vals
    ]
    norm = invs[0] + invs[1] + invs[2]

    bf16 = jnp.bfloat16
    w = [iv / norm for iv in invs]
    wh = [x.astype(bf16) for x in w]
    wl = [(x - y.astype(f32)).astype(bf16) for x, y in zip(w, wh)]
    wmat = jnp.concatenate(wh + wh + wl, axis=0)
    out_ref[:, :] = jax.lax.dot_general(
        sfpair_ref[:, :], wmat, (((1,), (0,)), ((), ())),
        preferred_element_type=f32,
    )


def _split_bf16(a):
    hi_f32 = jax.lax.reduce_precision(a, exponent_bits=8, mantissa_bits=7)
    hi = hi_f32.astype(jnp.bfloat16)
    lo = (a - hi_f32).astype(jnp.bfloat16)
    return hi, lo


@jax.jit
def kernel(xyz, sparse_xyz, sparse_frame):
    B, C, N = xyz.shape
    S = sparse_xyz.shape[2]
    TN = 512
    f32 = jnp.float32

    xq = xyz[0]
    sx = jnp.transpose(sparse_xyz[0])
    sf = jnp.transpose(sparse_frame[0])

    xh, xl = _split_bf16(xq)
    sxh, sxl = _split_bf16(sx)
    sfh, sfl = _split_bf16(sf)
    xsel = xh
    sxsel = sxh
    xpair = jnp.concatenate([xh, xl], axis=0)
    sxpair = jnp.concatenate([sxl, sxh], axis=1)
    sfpair = jnp.concatenate([sfh, sfl, sfh], axis=1)

    out = pl.pallas_call(
        _body,
        grid=(N // TN,),
        in_specs=[
            pl.BlockSpec((3, TN), lambda i: (0, i)),
            pl.BlockSpec((3, TN), lambda i: (0, i)),
            pl.BlockSpec((6, TN), lambda i: (0, i)),
            pl.BlockSpec((S, 3), lambda i: (0, 0)),
            pl.BlockSpec((S, 6), lambda i: (0, 0)),
            pl.BlockSpec((S, 9), lambda i: (0, 0)),
            pl.BlockSpec((S, 3), lambda i: (0, 0)),
        ],
        out_specs=pl.BlockSpec((S, TN), lambda i: (0, i)),
        out_shape=jax.ShapeDtypeStruct((S, N), f32),
        scratch_shapes=[pltpu.VMEM((S, 1), f32)],
    )(xq, xsel, xpair, sxsel, sxpair, sfpair, sx)
    return out[None]
